# baseline (speedup 1.0000x reference)
import jax
import jax.numpy as jnp
from jax.experimental import pallas as pl
from jax.experimental.pallas import tpu as pltpu

D = 2048
S = 2048
H = 16
DH = 128
NOPE = 64
RP = 64
QPD = 1024
KVPD = 1365
CKV_W = KVPD + RP
BQ = 256
EPS = 1e-5
SCALE = 1.0 / (DH ** 0.5)
LOG2E = 1.4426950408889634
F32 = jnp.float32
BF16 = jnp.bfloat16


def _rot_rope(x3):
    z = jnp.zeros_like(x3[..., :NOPE])
    return jnp.concatenate(
        [z, -x3[..., NOPE + RP // 2:], x3[..., NOPE:NOPE + RP // 2]], axis=-1)


def _proj_kernel(x_ref, wdq_ref, wuq_ref, qg_ref, qb_ref, cosq_ref, sinq_ref,
                 wdkv_ref, wukb_ref, wuv_ref, kvg_ref, kvb_ref,
                 cosk_ref, sink_ref, q_ref, ckv_ref, kb_ref, va_ref):
    xb = x_ref[...]

    cq = jnp.dot(xb, wdq_ref[...], preferred_element_type=F32)
    m = jnp.mean(cq, axis=-1, keepdims=True)
    dq = cq - m
    vq = jnp.mean(dq * dq, axis=-1, keepdims=True)
    cqn = dq * jax.lax.rsqrt(vq + EPS) * qg_ref[...] + qb_ref[...]
    q = jnp.dot(cqn.astype(BF16), wuq_ref[...], preferred_element_type=F32)
    q3 = q.reshape(BQ, H, DH)
    qh = (q3 * cosq_ref[...][:, None, :]
          + _rot_rope(q3) * sinq_ref[...][:, None, :])
    q_ref[...] = qh.reshape(BQ, D).astype(BF16)

    o = jnp.dot(xb, wdkv_ref[...], preferred_element_type=F32)
    kr = o[:, :DH]
    ckv = o[:, DH:]
    ckv_ref[...] = ckv
    mask = jax.lax.broadcasted_iota(jnp.int32, ckv.shape, 1) < KVPD
    cm = jnp.where(mask, ckv, 0.0)
    mk = jnp.sum(cm, axis=-1, keepdims=True) * (1.0 / KVPD)
    dk = jnp.where(mask, ckv - mk, 0.0)
    vk = jnp.sum(dk * dk, axis=-1, keepdims=True) * (1.0 / KVPD)
    kvn = ((ckv - mk) * jax.lax.rsqrt(vk + EPS) * kvg_ref[...]
           + kvb_ref[...]).astype(BF16)
    krr = kr * cosk_ref[...] + _rot_rope(kr) * sink_ref[...]
    kb = jnp.dot(kvn, wukb_ref[...], preferred_element_type=F32)
    kb = kb + jnp.concatenate([krr] * H, axis=-1)
    kb_ref[...] = kb.astype(BF16)
    va_ref[...] = jnp.dot(kvn, wuv_ref[...],
                          preferred_element_type=F32).astype(BF16)


def _attn_out_kernel(q_ref, kb_ref, va_ref, wo_ref, o_ref):
    h = pl.program_id(1)
    logits = jax.lax.dot_general(
        q_ref[...], kb_ref[...], (((1,), (1,)), ((), ())),
        preferred_element_type=F32)
    e = jnp.exp2(logits.astype(BF16))
    s = jnp.sum(e.astype(F32), axis=-1, keepdims=True)
    acc = jnp.dot(e, va_ref[...], preferred_element_type=F32)
    attn_b = (acc / s).astype(BF16)
    part = jax.lax.dot_general(
        attn_b, wo_ref[...], (((1,), (1,)), ((), ())),
        preferred_element_type=F32)

    @pl.when(h == 0)
    def _():
        o_ref[...] = part

    @pl.when(h > 0)
    def _():
        o_ref[...] += part


def kernel(x, W_dq, W_uq, q_ln_g, q_ln_b, W_dkv, W_ukv, kv_ln_g, kv_ln_b, W_o):
    x2 = x.reshape(S, D).astype(BF16)
    nI = S // BQ

    freqs = 1.0 / (10000.0 ** (jnp.arange(0, DH, 2, dtype=F32) / DH))
    emb = jnp.arange(S, dtype=F32)[:, None] * freqs[None, : RP // 2]
    cos64 = jnp.tile(jnp.cos(emb), (1, 2))
    sin64 = jnp.tile(jnp.sin(emb), (1, 2))
    ones64 = jnp.ones((S, NOPE), F32)
    zeros64 = jnp.zeros((S, NOPE), F32)
    qs = SCALE * LOG2E
    cosq = qs * jnp.concatenate([ones64, cos64], axis=-1)
    sinq = qs * jnp.concatenate([zeros64, sin64], axis=-1)
    cosk = jnp.concatenate([ones64, cos64], axis=-1)
    sink = jnp.concatenate([zeros64, sin64], axis=-1)

    wdq = W_dq.astype(BF16)
    wuq = W_uq.astype(BF16)
    wkr = jnp.pad(W_dkv[:, KVPD:], ((0, 0), (NOPE, 0)))
    wdkv_ext = jnp.concatenate([wkr, W_dkv], axis=-1).astype(BF16)
    w3 = W_ukv.reshape(KVPD, H, DH + NOPE)
    wukb = jnp.pad(w3[:, :, :NOPE],
                   ((0, RP), (0, 0), (0, RP))).reshape(CKV_W, H * DH)
    wukb = wukb.astype(BF16)
    wuv = jnp.pad(w3[:, :, NOPE:].reshape(KVPD, H * DH),
                  ((0, RP), (0, 0))).astype(BF16)
    wo = W_o.astype(BF16)
    kv_g = jnp.pad(kv_ln_g, (0, RP))[None, :]
    kv_b = jnp.pad(kv_ln_b, (0, RP))[None, :]

    Q, ckv, KB, VA = pl.pallas_call(
        _proj_kernel,
        grid=(nI,),
        in_specs=[
            pl.BlockSpec((BQ, D), lambda i: (i, 0)),
            pl.BlockSpec((D, QPD), lambda i: (0, 0)),
            pl.BlockSpec((QPD, D), lambda i: (0, 0)),
            pl.BlockSpec((1, QPD), lambda i: (0, 0)),
            pl.BlockSpec((1, QPD), lambda i: (0, 0)),
            pl.BlockSpec((BQ, DH), lambda i: (i, 0)),
            pl.BlockSpec((BQ, DH), lambda i: (i, 0)),
            pl.BlockSpec((D, DH + CKV_W), lambda i: (0, 0)),
            pl.BlockSpec((CKV_W, H * DH), lambda i: (0, 0)),
            pl.BlockSpec((CKV_W, H * DH), lambda i: (0, 0)),
            pl.BlockSpec((1, CKV_W), lambda i: (0, 0)),
            pl.BlockSpec((1, CKV_W), lambda i: (0, 0)),
            pl.BlockSpec((BQ, DH), lambda i: (i, 0)),
            pl.BlockSpec((BQ, DH), lambda i: (i, 0)),
        ],
        out_specs=[
            pl.BlockSpec((BQ, D), lambda i: (i, 0)),
            pl.BlockSpec((BQ, CKV_W), lambda i: (i, 0)),
            pl.BlockSpec((BQ, H * DH), lambda i: (i, 0)),
            pl.BlockSpec((BQ, H * DH), lambda i: (i, 0)),
        ],
        out_shape=[
            jax.ShapeDtypeStruct((S, D), BF16),
            jax.ShapeDtypeStruct((S, CKV_W), F32),
            jax.ShapeDtypeStruct((S, H * DH), BF16),
            jax.ShapeDtypeStruct((S, H * DH), BF16),
        ],
    )(x2, wdq, wuq, q_ln_g[None, :], q_ln_b[None, :], cosq, sinq,
      wdkv_ext, wukb, wuv, kv_g, kv_b, cosk, sink)

    out = pl.pallas_call(
        _attn_out_kernel,
        grid=(nI, H),
        in_specs=[
            pl.BlockSpec((BQ, DH), lambda i, h: (i, h)),
            pl.BlockSpec((S, DH), lambda i, h: (0, h)),
            pl.BlockSpec((S, DH), lambda i, h: (0, h)),
            pl.BlockSpec((D, DH), lambda i, h: (0, h)),
        ],
        out_specs=pl.BlockSpec((BQ, D), lambda i, h: (i, 0)),
        out_shape=jax.ShapeDtypeStruct((S, D), F32),
    )(Q, KB, VA, wo)

    return (out.reshape(1, S, D), ckv.reshape(1, S, CKV_W))


# speedup vs baseline: 1.1137x; 1.1137x over previous
import jax
import jax.numpy as jnp
from jax.experimental import pallas as pl
from jax.experimental.pallas import tpu as pltpu

D = 2048
S = 2048
H = 16
DH = 128
NOPE = 64
RP = 64
QPD = 1024
KVPD = 1365
CKV_W = KVPD + RP
BQ = 256
EPS = 1e-5
SCALE = 1.0 / (DH ** 0.5)
LOG2E = 1.4426950408889634
F32 = jnp.float32
BF16 = jnp.bfloat16


def _rot_rope(x3):
    z = jnp.zeros_like(x3[..., :NOPE])
    return jnp.concatenate(
        [z, -x3[..., NOPE + RP // 2:], x3[..., NOPE:NOPE + RP // 2]], axis=-1)


def _proj_kernel(x_ref, wdq_ref, wuq_ref, qg_ref, qb_ref, cosq_ref, sinq_ref,
                 wdkv_ref, wukb_ref, wuv_ref, kvg_ref, kvb_ref,
                 cosk_ref, sink_ref, q_ref, ckv_ref, kb_ref, va_ref):
    xb = x_ref[...]

    cq = jnp.dot(xb, wdq_ref[...], preferred_element_type=F32)
    m = jnp.mean(cq, axis=-1, keepdims=True)
    dq = cq - m
    vq = jnp.mean(dq * dq, axis=-1, keepdims=True)
    cqn = dq * jax.lax.rsqrt(vq + EPS) * qg_ref[...] + qb_ref[...]
    q = jnp.dot(cqn.astype(BF16), wuq_ref[...], preferred_element_type=F32)
    q3 = q.reshape(BQ, H, DH)
    qh = (q3 * cosq_ref[...][:, None, :]
          + _rot_rope(q3) * sinq_ref[...][:, None, :])
    q_ref[...] = qh.reshape(BQ, D).astype(BF16)

    o = jnp.dot(xb, wdkv_ref[...], preferred_element_type=F32)
    kr = o[:, :DH]
    ckv = o[:, DH:]
    ckv_ref[...] = ckv
    mask = jax.lax.broadcasted_iota(jnp.int32, ckv.shape, 1) < KVPD
    cm = jnp.where(mask, ckv, 0.0)
    mk = jnp.sum(cm, axis=-1, keepdims=True) * (1.0 / KVPD)
    dk = jnp.where(mask, ckv - mk, 0.0)
    vk = jnp.sum(dk * dk, axis=-1, keepdims=True) * (1.0 / KVPD)
    kvn = ((ckv - mk) * jax.lax.rsqrt(vk + EPS) * kvg_ref[...]
           + kvb_ref[...]).astype(BF16)
    krr = kr * cosk_ref[...] + _rot_rope(kr) * sink_ref[...]
    kb = jnp.dot(kvn, wukb_ref[...], preferred_element_type=F32)
    kb = kb + jnp.concatenate([krr] * H, axis=-1)
    kb_ref[...] = kb.astype(BF16)
    va_ref[...] = jnp.dot(kvn, wuv_ref[...],
                          preferred_element_type=F32).astype(BF16)


def _attn_kernel(q_ref, kb_ref, va_ref, o_ref):
    logits = jax.lax.dot_general(
        q_ref[...], kb_ref[...], (((1,), (1,)), ((), ())),
        preferred_element_type=F32)
    e = jnp.exp2(logits.astype(BF16))
    s = jnp.sum(e.astype(F32), axis=-1, keepdims=True)
    acc = jnp.dot(e, va_ref[...], preferred_element_type=F32)
    o_ref[...] = (acc / s).astype(BF16)


def _out_proj_kernel(a_ref, wo_ref, o_ref):
    o_ref[...] = jax.lax.dot_general(
        a_ref[...], wo_ref[...], (((1,), (1,)), ((), ())),
        preferred_element_type=F32)


def kernel(x, W_dq, W_uq, q_ln_g, q_ln_b, W_dkv, W_ukv, kv_ln_g, kv_ln_b, W_o):
    x2 = x.reshape(S, D).astype(BF16)
    nI = S // BQ

    freqs = 1.0 / (10000.0 ** (jnp.arange(0, DH, 2, dtype=F32) / DH))
    emb = jnp.arange(S, dtype=F32)[:, None] * freqs[None, : RP // 2]
    cos64 = jnp.tile(jnp.cos(emb), (1, 2))
    sin64 = jnp.tile(jnp.sin(emb), (1, 2))
    ones64 = jnp.ones((S, NOPE), F32)
    zeros64 = jnp.zeros((S, NOPE), F32)
    qs = SCALE * LOG2E
    cosq = qs * jnp.concatenate([ones64, cos64], axis=-1)
    sinq = qs * jnp.concatenate([zeros64, sin64], axis=-1)
    cosk = jnp.concatenate([ones64, cos64], axis=-1)
    sink = jnp.concatenate([zeros64, sin64], axis=-1)

    wdq = W_dq.astype(BF16)
    wuq = W_uq.astype(BF16)
    wkr = jnp.pad(W_dkv[:, KVPD:], ((0, 0), (NOPE, 0)))
    wdkv_ext = jnp.concatenate([wkr, W_dkv], axis=-1).astype(BF16)
    w3 = W_ukv.reshape(KVPD, H, DH + NOPE)
    wukb = jnp.pad(w3[:, :, :NOPE],
                   ((0, RP), (0, 0), (0, RP))).reshape(CKV_W, H * DH)
    wukb = wukb.astype(BF16)
    wuv = jnp.pad(w3[:, :, NOPE:].reshape(KVPD, H * DH),
                  ((0, RP), (0, 0))).astype(BF16)
    wo = W_o.astype(BF16)
    kv_g = jnp.pad(kv_ln_g, (0, RP))[None, :]
    kv_b = jnp.pad(kv_ln_b, (0, RP))[None, :]

    Q, ckv, KB, VA = pl.pallas_call(
        _proj_kernel,
        grid=(nI,),
        in_specs=[
            pl.BlockSpec((BQ, D), lambda i: (i, 0)),
            pl.BlockSpec((D, QPD), lambda i: (0, 0)),
            pl.BlockSpec((QPD, D), lambda i: (0, 0)),
            pl.BlockSpec((1, QPD), lambda i: (0, 0)),
            pl.BlockSpec((1, QPD), lambda i: (0, 0)),
            pl.BlockSpec((BQ, DH), lambda i: (i, 0)),
            pl.BlockSpec((BQ, DH), lambda i: (i, 0)),
            pl.BlockSpec((D, DH + CKV_W), lambda i: (0, 0)),
            pl.BlockSpec((CKV_W, H * DH), lambda i: (0, 0)),
            pl.BlockSpec((CKV_W, H * DH), lambda i: (0, 0)),
            pl.BlockSpec((1, CKV_W), lambda i: (0, 0)),
            pl.BlockSpec((1, CKV_W), lambda i: (0, 0)),
            pl.BlockSpec((BQ, DH), lambda i: (i, 0)),
            pl.BlockSpec((BQ, DH), lambda i: (i, 0)),
        ],
        out_specs=[
            pl.BlockSpec((BQ, D), lambda i: (i, 0)),
            pl.BlockSpec((BQ, CKV_W), lambda i: (i, 0)),
            pl.BlockSpec((BQ, H * DH), lambda i: (i, 0)),
            pl.BlockSpec((BQ, H * DH), lambda i: (i, 0)),
        ],
        out_shape=[
            jax.ShapeDtypeStruct((S, D), BF16),
            jax.ShapeDtypeStruct((S, CKV_W), F32),
            jax.ShapeDtypeStruct((S, H * DH), BF16),
            jax.ShapeDtypeStruct((S, H * DH), BF16),
        ],
    )(x2, wdq, wuq, q_ln_g[None, :], q_ln_b[None, :], cosq, sinq,
      wdkv_ext, wukb, wuv, kv_g, kv_b, cosk, sink)

    attn = pl.pallas_call(
        _attn_kernel,
        grid=(H, nI),
        in_specs=[
            pl.BlockSpec((BQ, DH), lambda h, i: (i, h)),
            pl.BlockSpec((S, DH), lambda h, i: (0, h)),
            pl.BlockSpec((S, DH), lambda h, i: (0, h)),
        ],
        out_specs=pl.BlockSpec((BQ, DH), lambda h, i: (i, h)),
        out_shape=jax.ShapeDtypeStruct((S, H * DH), BF16),
    )(Q, KB, VA)

    out = pl.pallas_call(
        _out_proj_kernel,
        grid=(nI,),
        in_specs=[
            pl.BlockSpec((BQ, D), lambda i: (i, 0)),
            pl.BlockSpec((D, D), lambda i: (0, 0)),
        ],
        out_specs=pl.BlockSpec((BQ, D), lambda i: (i, 0)),
        out_shape=jax.ShapeDtypeStruct((S, D), F32),
    )(attn, wo)

    return (out.reshape(1, S, D), ckv.reshape(1, S, CKV_W))


# speedup vs baseline: 1.2046x; 1.0816x over previous
import jax
import jax.numpy as jnp
from jax.experimental import pallas as pl
from jax.experimental.pallas import tpu as pltpu

D = 2048
S = 2048
H = 16
DH = 128
NOPE = 64
RP = 64
QPD = 1024
KVPD = 1365
CKV_W = KVPD + RP
BQ = 256
EPS = 1e-5
SCALE = 1.0 / (DH ** 0.5)
LOG2E = 1.4426950408889634
LOG2_10000 = 13.287712379549449
F32 = jnp.float32
BF16 = jnp.bfloat16


def _rot_rope(x3):
    z = jnp.zeros_like(x3[..., :NOPE])
    return jnp.concatenate(
        [z, -x3[..., NOPE + RP // 2:], x3[..., NOPE:NOPE + RP // 2]], axis=-1)


def _proj_kernel(x_ref, wdq_ref, wuq_ref, qg_ref, qb_ref,
                 wdkv_ref, wukb_ref, wuv_ref, kvg_ref, kvb_ref,
                 q_ref, ckv_ref, kb_ref, va_ref):
    xb = x_ref[...].astype(BF16)

    pos = (pl.program_id(0) * BQ
           + jax.lax.broadcasted_iota(jnp.int32, (BQ, RP // 2), 0)).astype(F32)
    j2 = jax.lax.broadcasted_iota(
        jnp.int32, (BQ, RP // 2), 1).astype(F32) * 2.0
    freq = jnp.exp2(-j2 * (LOG2_10000 / DH))
    emb = pos * freq
    c = jnp.cos(emb)
    si = jnp.sin(emb)
    ones = jnp.ones((BQ, NOPE), F32)
    zeros = jnp.zeros((BQ, NOPE), F32)
    cosk = jnp.concatenate([ones, c, c], axis=-1)
    sink = jnp.concatenate([zeros, si, si], axis=-1)
    qs = SCALE * LOG2E
    cosq = qs * cosk
    sinq = qs * sink

    cq = jnp.dot(xb, wdq_ref[...], preferred_element_type=F32)
    m = jnp.mean(cq, axis=-1, keepdims=True)
    dq = cq - m
    vq = jnp.mean(dq * dq, axis=-1, keepdims=True)
    cqn = dq * jax.lax.rsqrt(vq + EPS) * qg_ref[...] + qb_ref[...]
    q = jnp.dot(cqn.astype(BF16), wuq_ref[...], preferred_element_type=F32)
    q3 = q.reshape(BQ, H, DH)
    qh = (q3 * cosq[:, None, :]
          + _rot_rope(q3) * sinq[:, None, :])
    q_ref[...] = qh.reshape(BQ, D).astype(BF16)

    o = jnp.dot(xb, wdkv_ref[...], preferred_element_type=F32)
    kr = o[:, :DH]
    ckv = o[:, DH:]
    ckv_ref[...] = ckv
    mask = jax.lax.broadcasted_iota(jnp.int32, ckv.shape, 1) < KVPD
    cm = jnp.where(mask, ckv, 0.0)
    mk = jnp.sum(cm, axis=-1, keepdims=True) * (1.0 / KVPD)
    dk = jnp.where(mask, ckv - mk, 0.0)
    vk = jnp.sum(dk * dk, axis=-1, keepdims=True) * (1.0 / KVPD)
    kvn = ((ckv - mk) * jax.lax.rsqrt(vk + EPS) * kvg_ref[...]
           + kvb_ref[...]).astype(BF16)
    krr = kr * cosk + _rot_rope(kr) * sink
    kb = jnp.dot(kvn, wukb_ref[...], preferred_element_type=F32)
    kb = kb + jnp.concatenate([krr] * H, axis=-1)
    kb_ref[...] = kb.astype(BF16)
    va_ref[...] = jnp.dot(kvn, wuv_ref[...],
                          preferred_element_type=F32).astype(BF16)


def _attn_kernel(q_ref, kb_ref, va_ref, o_ref):
    logits = jax.lax.dot_general(
        q_ref[...], kb_ref[...], (((1,), (1,)), ((), ())),
        preferred_element_type=F32)
    e = jnp.exp2(logits.astype(BF16))
    s = jnp.sum(e.astype(F32), axis=-1, keepdims=True)
    acc = jnp.dot(e, va_ref[...], preferred_element_type=F32)
    o_ref[...] = (acc / s).astype(BF16)


def _out_proj_kernel(a_ref, wo_ref, o_ref, wobf_ref):
    @pl.when(pl.program_id(0) == 0)
    def _():
        wobf_ref[...] = wo_ref[...].astype(BF16)

    o_ref[...] = jax.lax.dot_general(
        a_ref[...], wobf_ref[...], (((1,), (1,)), ((), ())),
        preferred_element_type=F32)


def kernel(x, W_dq, W_uq, q_ln_g, q_ln_b, W_dkv, W_ukv, kv_ln_g, kv_ln_b, W_o):
    x2 = x.reshape(S, D)
    nI = S // BQ

    wdq = W_dq.astype(BF16)
    wuq = W_uq.astype(BF16)
    wkr = jnp.pad(W_dkv[:, KVPD:], ((0, 0), (NOPE, 0)))
    wdkv_ext = jnp.concatenate([wkr, W_dkv], axis=-1).astype(BF16)
    w3 = W_ukv.reshape(KVPD, H, DH + NOPE)
    wukb = jnp.pad(w3[:, :, :NOPE],
                   ((0, RP), (0, 0), (0, RP))).reshape(CKV_W, H * DH)
    wukb = wukb.astype(BF16)
    wuv = jnp.pad(w3[:, :, NOPE:].reshape(KVPD, H * DH),
                  ((0, RP), (0, 0))).astype(BF16)
    kv_g = jnp.pad(kv_ln_g, (0, RP))[None, :]
    kv_b = jnp.pad(kv_ln_b, (0, RP))[None, :]

    Q, ckv, KB, VA = pl.pallas_call(
        _proj_kernel,
        grid=(nI,),
        in_specs=[
            pl.BlockSpec((BQ, D), lambda i: (i, 0)),
            pl.BlockSpec((D, QPD), lambda i: (0, 0)),
            pl.BlockSpec((QPD, D), lambda i: (0, 0)),
            pl.BlockSpec((1, QPD), lambda i: (0, 0)),
            pl.BlockSpec((1, QPD), lambda i: (0, 0)),
            pl.BlockSpec((D, DH + CKV_W), lambda i: (0, 0)),
            pl.BlockSpec((CKV_W, H * DH), lambda i: (0, 0)),
            pl.BlockSpec((CKV_W, H * DH), lambda i: (0, 0)),
            pl.BlockSpec((1, CKV_W), lambda i: (0, 0)),
            pl.BlockSpec((1, CKV_W), lambda i: (0, 0)),
        ],
        out_specs=[
            pl.BlockSpec((BQ, D), lambda i: (i, 0)),
            pl.BlockSpec((BQ, CKV_W), lambda i: (i, 0)),
            pl.BlockSpec((BQ, H * DH), lambda i: (i, 0)),
            pl.BlockSpec((BQ, H * DH), lambda i: (i, 0)),
        ],
        out_shape=[
            jax.ShapeDtypeStruct((S, D), BF16),
            jax.ShapeDtypeStruct((S, CKV_W), F32),
            jax.ShapeDtypeStruct((S, H * DH), BF16),
            jax.ShapeDtypeStruct((S, H * DH), BF16),
        ],
    )(x2, wdq, wuq, q_ln_g[None, :], q_ln_b[None, :],
      wdkv_ext, wukb, wuv, kv_g, kv_b)

    attn = pl.pallas_call(
        _attn_kernel,
        grid=(H, nI),
        in_specs=[
            pl.BlockSpec((BQ, DH), lambda h, i: (i, h)),
            pl.BlockSpec((S, DH), lambda h, i: (0, h)),
            pl.BlockSpec((S, DH), lambda h, i: (0, h)),
        ],
        out_specs=pl.BlockSpec((BQ, DH), lambda h, i: (i, h)),
        out_shape=jax.ShapeDtypeStruct((S, H * DH), BF16),
    )(Q, KB, VA)

    out = pl.pallas_call(
        _out_proj_kernel,
        grid=(nI,),
        in_specs=[
            pl.BlockSpec((BQ, D), lambda i: (i, 0)),
            pl.BlockSpec((D, D), lambda i: (0, 0)),
        ],
        out_specs=pl.BlockSpec((BQ, D), lambda i: (i, 0)),
        out_shape=jax.ShapeDtypeStruct((S, D), F32),
        scratch_shapes=[pltpu.VMEM((D, D), BF16)],
    )(attn, W_o)

    return (out.reshape(1, S, D), ckv.reshape(1, S, CKV_W))
